# baseline (speedup 1.0000x reference)
import functools

import jax
import jax.numpy as jnp
from jax.experimental import pallas as pl
from jax.experimental.pallas import tpu as pltpu

N = 4096
D = 128
H = 64
BLK = 512
NB = N // BLK


def _gnn_kernel(x_ref, adj_ref, wc0, wc1, wc2, b0, b1, b2,
                out_ref, adj_bf, g_s, root_s, h_s):
    l = pl.program_id(0)
    i = pl.program_id(1)

    @pl.when(jnp.logical_and(l == 0, i == 0))
    def _():
        t = jax.lax.dot_general(x_ref[...], wc0[...],
                                (((1,), (0,)), ((), ())),
                                preferred_element_type=jnp.float32)
        g_s[...] = t[:, :H].astype(jnp.bfloat16)
        root_s[...] = t[:, H:]

    @pl.when(jnp.logical_and(l > 0, i == 0))
    def _():
        wc = jnp.where(l == 1, wc1[...], wc2[...])
        t = jax.lax.dot_general(h_s[...], wc,
                                (((1,), (0,)), ((), ())),
                                preferred_element_type=jnp.float32)
        g_s[...] = t[:, :H].astype(jnp.bfloat16)
        root_s[...] = t[:, H:]

    def step(a, bias):
        agg = jax.lax.dot_general(a, g_s[...],
                                  (((0,), (0,)), ((), ())),
                                  preferred_element_type=jnp.float32)
        res = jnp.maximum(agg + root_s[pl.ds(i * BLK, BLK), :] + bias, 0.0)
        out_ref[...] = res
        h_s[pl.ds(i * BLK, BLK), :] = res

    @pl.when(l == 0)
    def _():
        a = adj_ref[...].astype(jnp.bfloat16)
        adj_bf[i] = a
        step(a, b0[...])

    @pl.when(l == 1)
    def _():
        step(adj_bf[i], b1[...])

    @pl.when(l == 2)
    def _():
        step(adj_bf[i], b2[...])


@functools.partial(jax.jit, static_argnames=("interpret",))
def _run(X, adj_mat, W_rel0, b_rel0, W_root0, W_rel1, b_rel1, W_root1,
         W_rel2, b_rel2, W_root2, interpret=False):
    wc0 = jnp.concatenate([W_rel0.T, W_root0.T], axis=1)
    wc1 = jnp.concatenate([W_rel1.T, W_root1.T], axis=1)
    wc2 = jnp.concatenate([W_rel2.T, W_root2.T], axis=1)
    b0 = b_rel0.reshape(1, H)
    b1 = b_rel1.reshape(1, H)
    b2 = b_rel2.reshape(1, H)
    full = lambda shape: pl.BlockSpec(shape, lambda l, i: (0,) * len(shape))
    return pl.pallas_call(
        _gnn_kernel,
        grid=(3, NB),
        in_specs=[
            full((N, D)),
            pl.BlockSpec((N, BLK),
                         lambda l, i: (0, jnp.where(l == 0, i, 0))),
            full((D, 2 * H)), full((H, 2 * H)), full((H, 2 * H)),
            full((1, H)), full((1, H)), full((1, H)),
        ],
        out_specs=pl.BlockSpec((BLK, H), lambda l, i: (i, 0)),
        out_shape=jax.ShapeDtypeStruct((N, H), jnp.float32),
        scratch_shapes=[
            pltpu.VMEM((NB, N, BLK), jnp.bfloat16),
            pltpu.VMEM((N, H), jnp.bfloat16),
            pltpu.VMEM((N, H), jnp.float32),
            pltpu.VMEM((N, H), jnp.float32),
        ],
        interpret=interpret,
    )(X, adj_mat, wc0, wc1, wc2, b0, b1, b2)


def kernel(X, adj_mat, W_rel0, b_rel0, W_root0, W_rel1, b_rel1, W_root1,
           W_rel2, b_rel2, W_root2):
    return _run(X, adj_mat, W_rel0, b_rel0, W_root0, W_rel1, b_rel1, W_root1,
                W_rel2, b_rel2, W_root2)


# speedup vs baseline: 1.0263x; 1.0263x over previous
import functools

import jax
import jax.numpy as jnp
from jax.experimental import pallas as pl
from jax.experimental.pallas import tpu as pltpu

N = 4096
D = 128
H = 64
BLK = 512
NB = N // BLK


def _gnn_kernel(x_ref, adj_ref, wr0, br0, wo0, wr1, br1, wo1, wr2, br2, wo2,
                out_ref, adj_bf, h_s, g_s):
    l = pl.program_id(0)
    i = pl.program_id(1)

    @pl.when(jnp.logical_and(l == 0, i == 0))
    def _():
        g = jax.lax.dot_general(x_ref[...], wr0[...],
                                (((1,), (0,)), ((), ())),
                                preferred_element_type=jnp.float32)
        g_s[...] = g.astype(jnp.bfloat16)

    @pl.when(jnp.logical_and(l > 0, i == 0))
    def _():
        wr = jnp.where(l == 1, wr1[...], wr2[...])
        g = jax.lax.dot_general(h_s[...], wr,
                                (((1,), (0,)), ((), ())),
                                preferred_element_type=jnp.float32)
        g_s[...] = g.astype(jnp.bfloat16)

    @pl.when(l == 0)
    def _():
        a = adj_ref[...].astype(jnp.bfloat16)
        at = jax.lax.transpose(a, (1, 0))
        adj_bf[i] = at
        agg = jax.lax.dot_general(at, g_s[...],
                                  (((1,), (0,)), ((), ())),
                                  preferred_element_type=jnp.float32)
        x_blk = x_ref[pl.ds(i * BLK, BLK), :]
        root = jax.lax.dot_general(x_blk, wo0[...],
                                   (((1,), (0,)), ((), ())),
                                   preferred_element_type=jnp.float32)
        res = jnp.maximum(agg + root + br0[...], 0.0)
        out_ref[...] = res
        h_s[pl.ds(i * BLK, BLK), :] = res

    @pl.when(l > 0)
    def _():
        at = adj_bf[i]
        agg = jax.lax.dot_general(at, g_s[...],
                                  (((1,), (0,)), ((), ())),
                                  preferred_element_type=jnp.float32)
        wo = jnp.where(l == 1, wo1[...], wo2[...])
        br = jnp.where(l == 1, br1[...], br2[...])
        h_blk = h_s[pl.ds(i * BLK, BLK), :]
        root = jax.lax.dot_general(h_blk, wo,
                                   (((1,), (0,)), ((), ())),
                                   preferred_element_type=jnp.float32)
        res = jnp.maximum(agg + root + br, 0.0)
        out_ref[...] = res
        h_s[pl.ds(i * BLK, BLK), :] = res


@functools.partial(jax.jit, static_argnames=("interpret",))
def _run(X, adj_mat, W_rel0, b_rel0, W_root0, W_rel1, b_rel1, W_root1,
         W_rel2, b_rel2, W_root2, interpret=False):
    b0 = b_rel0.reshape(1, H)
    b1 = b_rel1.reshape(1, H)
    b2 = b_rel2.reshape(1, H)
    full = lambda shape: pl.BlockSpec(shape, lambda l, i: (0,) * len(shape))
    return pl.pallas_call(
        _gnn_kernel,
        grid=(3, NB),
        in_specs=[
            full((N, D)),
            pl.BlockSpec((N, BLK),
                         lambda l, i: (0, jnp.where(l == 0, i, 0))),
            full((D, H)), full((1, H)), full((D, H)),
            full((H, H)), full((1, H)), full((H, H)),
            full((H, H)), full((1, H)), full((H, H)),
        ],
        out_specs=pl.BlockSpec((BLK, H), lambda l, i: (i, 0)),
        out_shape=jax.ShapeDtypeStruct((N, H), jnp.float32),
        scratch_shapes=[
            pltpu.VMEM((NB, BLK, N), jnp.bfloat16),
            pltpu.VMEM((N, H), jnp.float32),
            pltpu.VMEM((N, H), jnp.bfloat16),
        ],
        interpret=interpret,
    )(X, adj_mat, W_rel0.T, b0, W_root0.T, W_rel1.T, b1, W_root1.T,
      W_rel2.T, b2, W_root2.T)


def kernel(X, adj_mat, W_rel0, b_rel0, W_root0, W_rel1, b_rel1, W_root1,
           W_rel2, b_rel2, W_root2):
    return _run(X, adj_mat, W_rel0, b_rel0, W_root0, W_rel1, b_rel1, W_root1,
                W_rel2, b_rel2, W_root2)


# speedup vs baseline: 1.1733x; 1.1432x over previous
import functools

import jax
import jax.numpy as jnp
from jax.experimental import pallas as pl
from jax.experimental.pallas import tpu as pltpu

N = 4096
D = 128
H = 64
BLK = 512
NB = N // BLK


def _gnn_kernel(x_ref, adj_ref, wr0, br0, wo0, wr1, br1, wo1, wr2, br2, wo2,
                out_ref, adj_bf, h_s, g_s):
    l = pl.program_id(0)
    i = pl.program_id(1)

    @pl.when(jnp.logical_and(l == 0, i == 0))
    def _():
        g = jax.lax.dot_general(x_ref[...], wr0[...],
                                (((1,), (1,)), ((), ())),
                                preferred_element_type=jnp.float32)
        g_s[...] = g.astype(jnp.bfloat16)

    @pl.when(jnp.logical_and(l > 0, i == 0))
    def _():
        wr = jnp.where(l == 1, wr1[...], wr2[...])
        g = jax.lax.dot_general(h_s[...], wr,
                                (((1,), (1,)), ((), ())),
                                preferred_element_type=jnp.float32)
        g_s[...] = g.astype(jnp.bfloat16)

    @pl.when(l == 0)
    def _():
        a = adj_ref[...].astype(jnp.bfloat16)
        adj_bf[i] = a
        agg = jax.lax.dot_general(a, g_s[...],
                                  (((0,), (0,)), ((), ())),
                                  preferred_element_type=jnp.float32)
        x_blk = x_ref[pl.ds(i * BLK, BLK), :]
        root = jax.lax.dot_general(x_blk, wo0[...],
                                   (((1,), (1,)), ((), ())),
                                   preferred_element_type=jnp.float32)
        res = jnp.maximum(agg + root + br0[...], 0.0)
        out_ref[...] = res
        h_s[pl.ds(i * BLK, BLK), :] = res

    @pl.when(l > 0)
    def _():
        a = adj_bf[i]
        agg = jax.lax.dot_general(a, g_s[...],
                                  (((0,), (0,)), ((), ())),
                                  preferred_element_type=jnp.float32)
        wo = jnp.where(l == 1, wo1[...], wo2[...])
        br = jnp.where(l == 1, br1[...], br2[...])
        h_blk = h_s[pl.ds(i * BLK, BLK), :]
        root = jax.lax.dot_general(h_blk, wo,
                                   (((1,), (1,)), ((), ())),
                                   preferred_element_type=jnp.float32)
        res = jnp.maximum(agg + root + br, 0.0)
        out_ref[...] = res
        h_s[pl.ds(i * BLK, BLK), :] = res


@functools.partial(jax.jit, static_argnames=("interpret",))
def _run(X, adj_mat, W_rel0, b_rel0, W_root0, W_rel1, b_rel1, W_root1,
         W_rel2, b_rel2, W_root2, interpret=False):
    b0 = b_rel0.reshape(1, H)
    b1 = b_rel1.reshape(1, H)
    b2 = b_rel2.reshape(1, H)
    full = lambda shape: pl.BlockSpec(shape, lambda l, i: (0,) * len(shape))
    return pl.pallas_call(
        _gnn_kernel,
        grid=(3, NB),
        in_specs=[
            full((N, D)),
            pl.BlockSpec((N, BLK),
                         lambda l, i: (0, jnp.where(l == 0, i, 0))),
            full((H, D)), full((1, H)), full((H, D)),
            full((H, H)), full((1, H)), full((H, H)),
            full((H, H)), full((1, H)), full((H, H)),
        ],
        out_specs=pl.BlockSpec((BLK, H), lambda l, i: (i, 0)),
        out_shape=jax.ShapeDtypeStruct((N, H), jnp.float32),
        scratch_shapes=[
            pltpu.VMEM((NB, N, BLK), jnp.bfloat16),
            pltpu.VMEM((N, H), jnp.float32),
            pltpu.VMEM((N, H), jnp.bfloat16),
        ],
        interpret=interpret,
    )(X, adj_mat, W_rel0, b0, W_root0, W_rel1, b1, W_root1, W_rel2, b2, W_root2)


def kernel(X, adj_mat, W_rel0, b_rel0, W_root0, W_rel1, b_rel1, W_root1,
           W_rel2, b_rel2, W_root2):
    return _run(X, adj_mat, W_rel0, b_rel0, W_root0, W_rel1, b_rel1, W_root1,
                W_rel2, b_rel2, W_root2)


# speedup vs baseline: 1.3033x; 1.1108x over previous
import functools

import jax
import jax.numpy as jnp
from jax.experimental import pallas as pl
from jax.experimental.pallas import tpu as pltpu

N = 4096
D = 128
H = 64
BLK = 512
NB = N // BLK


def _gnn_kernel(x_ref, adj_ref, wr0, br0, wo0, wr1, br1, wo1, wr2, br2, wo2,
                out_ref, adj_bf, h_s, g_s):
    s = pl.program_id(0)

    @pl.when(s == 0)
    def _():
        g = jax.lax.dot_general(x_ref[...], wr0[...],
                                (((1,), (1,)), ((), ())),
                                preferred_element_type=jnp.float32)
        g_s[...] = g.astype(jnp.bfloat16)

    @pl.when(s < NB)
    def _():
        a = adj_ref[...].astype(jnp.bfloat16)
        adj_bf[s] = a
        agg = jax.lax.dot_general(a, g_s[...],
                                  (((0,), (0,)), ((), ())),
                                  preferred_element_type=jnp.float32)
        x_blk = x_ref[pl.ds(s * BLK, BLK), :]
        root = jax.lax.dot_general(x_blk, wo0[...],
                                   (((1,), (1,)), ((), ())),
                                   preferred_element_type=jnp.float32)
        res = jnp.maximum(agg + root + br0[...], 0.0)
        h_s[pl.ds(s * BLK, BLK), :] = res

    def layer(wr, br, wo, last):
        g = jax.lax.dot_general(h_s[...], wr[...],
                                (((1,), (1,)), ((), ())),
                                preferred_element_type=jnp.float32)
        g_s[...] = g.astype(jnp.bfloat16)
        for i in range(NB):
            agg = jax.lax.dot_general(adj_bf[i], g_s[...],
                                      (((0,), (0,)), ((), ())),
                                      preferred_element_type=jnp.float32)
            h_blk = h_s[i * BLK:(i + 1) * BLK, :]
            root = jax.lax.dot_general(h_blk, wo[...],
                                       (((1,), (1,)), ((), ())),
                                       preferred_element_type=jnp.float32)
            res = jnp.maximum(agg + root + br[...], 0.0)
            if last:
                out_ref[i * BLK:(i + 1) * BLK, :] = res
            else:
                h_s[i * BLK:(i + 1) * BLK, :] = res

    @pl.when(s == NB)
    def _():
        layer(wr1, br1, wo1, last=False)

    @pl.when(s == NB + 1)
    def _():
        layer(wr2, br2, wo2, last=True)


@functools.partial(jax.jit, static_argnames=("interpret",))
def _run(X, adj_mat, W_rel0, b_rel0, W_root0, W_rel1, b_rel1, W_root1,
         W_rel2, b_rel2, W_root2, interpret=False):
    b0 = b_rel0.reshape(1, H)
    b1 = b_rel1.reshape(1, H)
    b2 = b_rel2.reshape(1, H)
    full = lambda shape: pl.BlockSpec(shape, lambda s: (0,) * len(shape))
    return pl.pallas_call(
        _gnn_kernel,
        grid=(NB + 2,),
        in_specs=[
            full((N, D)),
            pl.BlockSpec((N, BLK),
                         lambda s: (0, jnp.minimum(s, NB - 1))),
            full((H, D)), full((1, H)), full((H, D)),
            full((H, H)), full((1, H)), full((H, H)),
            full((H, H)), full((1, H)), full((H, H)),
        ],
        out_specs=full((N, H)),
        out_shape=jax.ShapeDtypeStruct((N, H), jnp.float32),
        scratch_shapes=[
            pltpu.VMEM((NB, N, BLK), jnp.bfloat16),
            pltpu.VMEM((N, H), jnp.float32),
            pltpu.VMEM((N, H), jnp.bfloat16),
        ],
        interpret=interpret,
    )(X, adj_mat, W_rel0, b0, W_root0, W_rel1, b1, W_root1, W_rel2, b2, W_root2)


def kernel(X, adj_mat, W_rel0, b_rel0, W_root0, W_rel1, b_rel1, W_root1,
           W_rel2, b_rel2, W_root2):
    return _run(X, adj_mat, W_rel0, b_rel0, W_root0, W_rel1, b_rel1, W_root1,
                W_rel2, b_rel2, W_root2)
